# Initial kernel scaffold; baseline (speedup 1.0000x reference)
import jax
import jax.numpy as jnp
from jax.experimental import pallas as pl


def kernel(xyz, points, affine_alpha, affine_beta):
    raise NotImplementedError("write your pallas kernel here")


# speedup vs baseline: 12.0142x; 12.0142x over previous
import functools

import jax
import jax.numpy as jnp
import numpy as np
from jax import lax
from jax.experimental import pallas as pl
from jax.experimental.pallas import tpu as pltpu
from jax.experimental.pallas import tpu_sc as plsc

_RADIUS = 0.2
_NSAMPLE = 32
_REDUCE = 4

_INTERPRET = False


def _fps_body(xt_ref, idx_ref, nxyz_ref):
    _, B, N = xt_ref.shape
    S = idx_ref.shape[1]
    x = xt_ref[0]
    y = xt_ref[1]
    z = xt_ref[2]
    iota = jax.lax.broadcasted_iota(jnp.int32, (B, N), 1)
    lane = jax.lax.broadcasted_iota(jnp.int32, (B, 128), 1)

    def body(t, carry):
        dists, far, acci, accx, accy, accz = carry
        ins = lane == t
        acci = jnp.where(ins, far, acci)
        m = iota == far
        cx = jnp.sum(jnp.where(m, x, 0.0), axis=1, keepdims=True)
        cy = jnp.sum(jnp.where(m, y, 0.0), axis=1, keepdims=True)
        cz = jnp.sum(jnp.where(m, z, 0.0), axis=1, keepdims=True)
        accx = jnp.where(ins, cx, accx)
        accy = jnp.where(ins, cy, accy)
        accz = jnp.where(ins, cz, accz)
        dx = x - cx
        dy = y - cy
        dz = z - cz
        d = dx * dx + dy * dy + dz * dz
        dists = jnp.minimum(dists, d)
        mx = jnp.max(dists, axis=1, keepdims=True)
        far = jnp.min(jnp.where(dists == mx, iota, N), axis=1, keepdims=True)
        return dists, far, acci, accx, accy, accz

    dists = jnp.full((B, N), 1e10, dtype=jnp.float32)
    far = jnp.zeros((B, 1), dtype=jnp.int32)
    zi = jax.lax.broadcasted_iota(jnp.int32, (B, 128), 0) * jax.lax.broadcasted_iota(jnp.int32, (B, 128), 1)
    zf = zi.astype(jnp.float32)
    carry = (dists, far, zi, zf, zf, zf)
    for j in range(S // 128):
        carry = jax.lax.fori_loop(0, 128, body, carry)
        dists, far, acci, accx, accy, accz = carry
        idx_ref[:, j * 128:(j + 1) * 128] = acci
        nxyz_ref[0, :, j * 128:(j + 1) * 128] = accx
        nxyz_ref[1, :, j * 128:(j + 1) * 128] = accy
        nxyz_ref[2, :, j * 128:(j + 1) * 128] = accz


def _fps_pallas(xt, S):
    _, B, N = xt.shape
    return pl.pallas_call(
        _fps_body,
        out_shape=(
            jax.ShapeDtypeStruct((B, S), jnp.int32),
            jax.ShapeDtypeStruct((3, B, S), jnp.float32),
        ),
        interpret=_INTERPRET,
    )(xt)


_QT = 256
_BIG = 4096.0


def _ballq_body(q_ref, s_ref, qn_ref, sn_ref, o_ref, cand_ref):
    QT = q_ref.shape[1]
    N = s_ref.shape[1]
    K = _NSAMPLE
    r2 = np.float32(_RADIUS * _RADIUS)
    q = q_ref[0]
    s = s_ref[0]
    qn = qn_ref[0]
    sn = sn_ref[0]
    dot = jax.lax.dot_general(q, s, (((1,), (1,)), ((), ())),
                              preferred_element_type=jnp.float32)
    sqd = (qn + sn) - 2.0 * dot
    iotaf = jax.lax.broadcasted_iota(jnp.int32, (QT, N), 1).astype(jnp.float32)
    cand_ref[...] = jnp.where(sqd < r2, iotaf, _BIG)
    lane = jax.lax.broadcasted_iota(jnp.int32, (QT, K), 1)

    def ext(j, carry):
        mnp, acc = carry
        c = cand_ref[...]
        c = jnp.where(c == mnp, _BIG, c)
        cand_ref[...] = c
        mn = jnp.min(c, axis=1, keepdims=True)
        acc = jnp.where(lane == j, mn, acc)
        return mn, acc

    mn0 = qn * 0.0 - 1.0
    acc0 = jnp.broadcast_to(qn, (QT, K)) + lane.astype(jnp.float32)
    _, acc = jax.lax.fori_loop(0, K, ext, (mn0, acc0))
    first = acc[:, 0:1]
    first = jnp.where(first == _BIG, 0.0, first)
    accf = jnp.where(acc == _BIG, first, acc)
    o_ref[0] = accf.astype(jnp.int32)


def _ballq_pallas(new_xyz, xyz, qn, sn):
    B, S, _ = new_xyz.shape
    N = xyz.shape[1]
    nt = S // _QT
    return pl.pallas_call(
        _ballq_body,
        grid=(B, nt),
        in_specs=[
            pl.BlockSpec((1, _QT, 3), lambda b, t: (b, t, 0)),
            pl.BlockSpec((1, N, 3), lambda b, t: (b, 0, 0)),
            pl.BlockSpec((1, _QT, 1), lambda b, t: (b, t, 0)),
            pl.BlockSpec((1, 1, N), lambda b, t: (b, 0, 0)),
        ],
        out_specs=pl.BlockSpec((1, _QT, _NSAMPLE), lambda b, t: (b, t, 0)),
        out_shape=jax.ShapeDtypeStruct((B, S, _NSAMPLE), jnp.int32),
        scratch_shapes=[pltpu.VMEM((_QT, N), jnp.float32)],
        interpret=_INTERPRET,
    )(new_xyz, xyz, qn, sn)


def _gatherstats_sc(pts_rows, idxf, fpsf, B, N, C):
    RK = idxf.shape[0]
    R = fpsf.shape[0]
    K = _NSAMPLE
    NW = 32
    rk_w = RK // NW
    q_w = R // NW
    wpb = NW // B
    CH = 128
    nch = rk_w // CH
    mesh = plsc.VectorSubcoreMesh(core_axis_name="c", subcore_axis_name="s")

    @functools.partial(
        pl.kernel, mesh=mesh,
        out_type=(
            jax.ShapeDtypeStruct((RK, C), jnp.float32),
            jax.ShapeDtypeStruct((R * C,), jnp.float32),
            jax.ShapeDtypeStruct((R, C), jnp.float32),
            jax.ShapeDtypeStruct((NW, 16), jnp.float32),
        ),
        scratch_types=[
            pltpu.VMEM((CH,), jnp.int32),
            pltpu.VMEM((CH, 128), jnp.float32),
            pltpu.VMEM((256 * 128,), jnp.float32),
            pltpu.VMEM((16,), jnp.float32),
            pltpu.SemaphoreType.DMA,
        ],
    )
    def k(pts_hbm, idx_hbm, fps_hbm, gat_hbm, ksum_hbm, newp_hbm, sq_hbm,
          idxv, rows, ksb, sqv, sem):
        wid = lax.axis_index("s") * 2 + lax.axis_index("c")
        b = wid // wpb
        bN = b * N
        ro = wid * rk_w
        qo = wid * q_w
        zero16 = jnp.zeros((16,), jnp.float32)

        def zb(i, _):
            ksb[pl.ds(i * 16, 16)] = zero16
            return 0
        lax.fori_loop(0, (q_w * C) // 16, zb, 0)

        def chunk(t, sqacc):
            pltpu.sync_copy(idx_hbm.at[pl.ds(ro + t * CH, CH)], idxv)

            def addb(u, _):
                idxv[pl.ds(u * 16, 16)] = idxv[pl.ds(u * 16, 16)] + bN
                return 0
            lax.fori_loop(0, CH // 16, addb, 0)

            pltpu.async_copy(pts_hbm.at[idxv], rows, sem).wait()
            pltpu.sync_copy(rows, gat_hbm.at[pl.ds(ro + t * CH, CH)])

            def row_acc(r, sqa):
                qrow = t * (CH // K) + r // K

                def lane_acc(l, sqa2):
                    v = rows[r, pl.ds(l * 16, 16)]
                    a = ksb[pl.ds(qrow * C + l * 16, 16)]
                    ksb[pl.ds(qrow * C + l * 16, 16)] = a + v
                    return sqa2 + v * v
                return lax.fori_loop(0, C // 16, lane_acc, sqa)
            return lax.fori_loop(0, CH, row_acc, sqacc)

        sqacc = lax.fori_loop(0, nch, chunk, zero16)
        sqv[pl.ds(0, 16)] = sqacc
        pltpu.sync_copy(sqv, sq_hbm.at[wid])
        pltpu.sync_copy(ksb, ksum_hbm.at[pl.ds(qo * C, q_w * C)])

        def npchunk(t, _):
            pltpu.sync_copy(fps_hbm.at[pl.ds(qo + t * CH, CH)], idxv)

            def addb(u, _):
                idxv[pl.ds(u * 16, 16)] = idxv[pl.ds(u * 16, 16)] + bN
                return 0
            lax.fori_loop(0, CH // 16, addb, 0)
            pltpu.async_copy(pts_hbm.at[idxv], rows, sem).wait()
            pltpu.sync_copy(rows, newp_hbm.at[pl.ds(qo + t * CH, CH)])
            return 0
        lax.fori_loop(0, q_w // CH, npchunk, 0)

    return k(pts_rows, idxf, fpsf)


def _std_body(ks_ref, sq_ref, o_ref):
    B = ks_ref.shape[0]
    ks = ks_ref[...]
    sq = sq_ref[...]
    M = ks.shape[1] * _NSAMPLE * ks.shape[2]
    sumsq_g = jnp.sum(sq, axis=1, keepdims=True)
    ks2 = jnp.sum(ks * ks, axis=2)
    sum_ks2 = jnp.sum(ks2, axis=1, keepdims=True)
    tot = sumsq_g - sum_ks2 * (1.0 / _NSAMPLE)
    std = jnp.sqrt(tot * (1.0 / (M - 1)))
    o_ref[...] = jnp.broadcast_to((std + 1e-5)[:, :, None], (B, 1, 128))


def _std_pallas(ksum3, sqpart):
    B = ksum3.shape[0]
    return pl.pallas_call(
        _std_body,
        out_shape=jax.ShapeDtypeStruct((B, 1, 128), jnp.float32),
        interpret=_INTERPRET,
    )(ksum3, sqpart)


_FT = 64


def _fin_body(gat_ref, ks_ref, np_ref, std_ref, al_ref, be_ref, o_ref):
    K = _NSAMPLE
    C = gat_ref.shape[1]
    g = gat_ref[...].reshape(_FT, K, C)
    mean = ks_ref[...] * (1.0 / K)
    y = g - mean[:, None, :]
    stdb = std_ref[0, 0, 0]
    al = al_ref[0]
    be = be_ref[0]
    left = (y / stdb) * al + be
    o_ref[:, 0:C] = left.reshape(_FT * K, C)
    rep = jnp.broadcast_to(np_ref[...][:, None, :], (_FT, K, C))
    o_ref[:, C:2 * C] = rep.reshape(_FT * K, C)


def _finalize_pallas(gat, ksum, newp, stdb, alpha2, beta2, B):
    RK, C = gat.shape
    R = ksum.shape[0]
    K = _NSAMPLE
    nt = R // _FT
    tpb = nt // B
    return pl.pallas_call(
        _fin_body,
        grid=(nt,),
        in_specs=[
            pl.BlockSpec((_FT * K, C), lambda t: (t, 0)),
            pl.BlockSpec((_FT, C), lambda t: (t, 0)),
            pl.BlockSpec((_FT, C), lambda t: (t, 0)),
            pl.BlockSpec((1, 1, 128), lambda t: (t // tpb, 0, 0)),
            pl.BlockSpec((1, C), lambda t: (0, 0)),
            pl.BlockSpec((1, C), lambda t: (0, 0)),
        ],
        out_specs=pl.BlockSpec((_FT * K, 2 * C), lambda t: (t, 0)),
        out_shape=jax.ShapeDtypeStruct((RK, 2 * C), jnp.float32),
        interpret=_INTERPRET,
    )(gat, ksum, newp, stdb, alpha2, beta2)


def kernel(xyz, points, affine_alpha, affine_beta):
    B, N, _ = xyz.shape
    C = points.shape[1]
    S = N // _REDUCE
    K = _NSAMPLE

    xt = jnp.transpose(xyz, (2, 0, 1))
    fps_idx, nxyz3 = _fps_pallas(xt, S)
    new_xyz = jnp.transpose(nxyz3, (1, 2, 0))

    qn = jnp.sum(new_xyz ** 2, axis=-1)[:, :, None]
    sn = jnp.sum(xyz ** 2, axis=-1)[:, None, :]
    idx = _ballq_pallas(new_xyz, xyz, qn, sn)
    idxf = idx.reshape(B * S * K)

    pts_rows = jnp.transpose(points, (0, 2, 1)).reshape(B * N, C)
    gat, ksum, newp, sqpart = _gatherstats_sc(
        pts_rows, idxf, fps_idx.reshape(-1), B, N, C)

    stdb = _std_pallas(ksum.reshape(B, S, C), sqpart.reshape(B, (32 // B) * 16))

    out = _finalize_pallas(gat, ksum.reshape(B * S, C), newp, stdb,
                           affine_alpha.reshape(1, C), affine_beta.reshape(1, C), B)
    return (new_xyz, out.reshape(B, S, K, 2 * C))
